# baseline (speedup 1.0000x reference)
import jax
import jax.numpy as jnp
from jax import lax
from jax.experimental import pallas as pl
from jax.experimental.pallas import tpu as pltpu
from jax.experimental.pallas import tpu_sc as plsc

B = 4096
D = 32
W = 10
NNEG = 50
NPAIR = W + NNEG
RPK = 128 // D

NC, NS = 2, 16
NW = NC * NS
BPW = B // NW
G = 16
NGROUP = BPW // G
NCHUNK = NGROUP * 6
CW = G * W


def _sc_scores_body(tgt_o_hbm, tgt_r4_hbm, ctx_o_hbm, ctx_r4_hbm,
                    neg_o_hbm, neg_r4_hbm, emb_in_hbm, emb_out_hbm,
                    scores_hbm,
                    io0, io1, ir0, ir1, rows0, rows1,
                    tio0, tio1, tir0, tir1, trows0, trows1,
                    score_v, sem_i0, sem_i1, sem_g0, sem_g1):
    wid = lax.axis_index("s") * NC + lax.axis_index("c")
    base = wid * BPW

    io = (io0, io1)
    ir = (ir0, ir1)
    rows = (rows0, rows1)
    tio = (tio0, tio1)
    tir = (tir0, tir1)
    trows = (trows0, trows1)
    sem_i = (sem_i0, sem_i1)
    sem_g = (sem_g0, sem_g1)

    def idx_copies(s, p):
        g, c = divmod(s, 6)
        b0 = base + g * G
        if c == 0:
            off = b0 * W
        else:
            off = ((c - 1) * B + b0) * W
        osrc = ctx_o_hbm if c == 0 else neg_o_hbm
        rsrc = ctx_r4_hbm if c == 0 else neg_r4_hbm
        out = [
            (osrc.at[pl.ds(off, CW)], io[p]),
            (rsrc.at[pl.ds(off, CW)], ir[p]),
        ]
        if c == 0:
            gp = g & 1
            out.append((tgt_o_hbm.at[pl.ds(b0, G)], tio[gp]))
            out.append((tgt_r4_hbm.at[pl.ds(b0, G)], tir[gp]))
        return out

    def gather_copies(s, p):
        g, c = divmod(s, 6)
        out = []
        for off in range(0, CW, 128):
            sz = min(128, CW - off)
            out.append((emb_out_hbm.at[ir[p].at[pl.ds(off, sz)]],
                        rows[p].at[pl.ds(off, sz), :]))
        if c == 0:
            gp = g & 1
            out.append((emb_in_hbm.at[tir[gp]], trows[gp]))
        return out

    def fire(pairs, sem):
        for s, d in pairs:
            pltpu.async_copy(s, d, sem)

    def drain(pairs, sem):
        for s, d in pairs:
            pltpu.make_async_copy(s, d, sem).wait()

    lane = lax.iota(jnp.int32, 16)
    lane10 = lane * W

    def compute(s, p):
        g, c = divmod(s, 6)
        gp = g & 1
        tcol = (tio[gp][...] & (RPK - 1)) * D
        colb = [(plsc.load_gather(io[p], [lane10 + j]) & (RPK - 1)) * D
                for j in range(W)]

        def body(d, accs):
            tv = plsc.load_gather(trows[gp], [lane, tcol + d])
            return tuple(
                accs[j] + tv * plsc.load_gather(rows[p],
                                                [lane10 + j, colb[j] + d])
                for j in range(W))

        accs = lax.fori_loop(
            0, D, body,
            tuple(jnp.zeros((16,), jnp.float32) for _ in range(W)))
        for j in range(W):
            val = accs[j] if c == 0 else -accs[j]
            score_v[c * W + j, pl.ds(g * G, G)] = val

    fire(idx_copies(0, 0), sem_i[0])
    fire(idx_copies(1, 1), sem_i[1])
    drain(idx_copies(0, 0), sem_i[0])
    fire(gather_copies(0, 0), sem_g[0])
    for s in range(NCHUNK):
        p = s & 1
        if s + 1 < NCHUNK:
            drain(idx_copies(s + 1, 1 - p), sem_i[1 - p])
            fire(gather_copies(s + 1, 1 - p), sem_g[1 - p])
        drain(gather_copies(s, p), sem_g[p])
        if s + 2 < NCHUNK:
            fire(idx_copies(s + 2, p), sem_i[p])
        compute(s, p)
    pltpu.sync_copy(score_v, scores_hbm.at[:, pl.ds(base, BPW)])


_sc_scores = pl.kernel(
    out_type=jax.ShapeDtypeStruct((NPAIR, B), jnp.float32),
    mesh=plsc.VectorSubcoreMesh(core_axis_name="c", subcore_axis_name="s"),
    compiler_params=pltpu.CompilerParams(needs_layout_passes=False),
    scratch_types=[
        pltpu.VMEM((CW,), jnp.int32), pltpu.VMEM((CW,), jnp.int32),
        pltpu.VMEM((CW,), jnp.int32), pltpu.VMEM((CW,), jnp.int32),
        pltpu.VMEM((CW, 128), jnp.float32), pltpu.VMEM((CW, 128), jnp.float32),
        pltpu.VMEM((G,), jnp.int32), pltpu.VMEM((G,), jnp.int32),
        pltpu.VMEM((G,), jnp.int32), pltpu.VMEM((G,), jnp.int32),
        pltpu.VMEM((G, 128), jnp.float32), pltpu.VMEM((G, 128), jnp.float32),
        pltpu.VMEM((NPAIR, BPW), jnp.float32),
        pltpu.SemaphoreType.DMA, pltpu.SemaphoreType.DMA,
        pltpu.SemaphoreType.DMA, pltpu.SemaphoreType.DMA,
    ],
)(_sc_scores_body)


def _tc_loss_body(s_ref, o_ref):
    x = s_ref[...]
    o_ref[0, 0] = -jnp.sum(jax.nn.log_sigmoid(x)) / (B * W)


_tc_loss = pl.pallas_call(
    _tc_loss_body,
    out_shape=jax.ShapeDtypeStruct((1, 1), jnp.float32),
    out_specs=pl.BlockSpec(memory_space=pltpu.SMEM),
)


def kernel(target, context, negative_samples, emb_in, emb_out):
    tgt_o = target.reshape(-1).astype(jnp.int32)
    ctx_o = context.reshape(-1).astype(jnp.int32)
    neg_o = (negative_samples.astype(jnp.int32)
             .reshape(B, NNEG // W, W).transpose(1, 0, 2).reshape(-1))
    tgt_r4 = tgt_o // RPK
    ctx_r4 = ctx_o // RPK
    neg_r4 = neg_o // RPK
    emb_in4 = emb_in.reshape(-1, 128)
    emb_out4 = emb_out.reshape(-1, 128)
    scores = _sc_scores(tgt_o, tgt_r4, ctx_o, ctx_r4, neg_o, neg_r4,
                        emb_in4, emb_out4)
    return _tc_loss(scores)[0, 0]
